# baseline (speedup 1.0000x reference)
import functools

import jax
import jax.numpy as jnp
from jax import lax
from jax.experimental import pallas as pl
from jax.experimental.pallas import tpu as pltpu
from jax.experimental.pallas import tpu_sc as plsc


def _mlp1_body(x_ref, w1_ref, b1_ref, h_ref, s_ref, ss_ref):
    h = jnp.dot(x_ref[...], w1_ref[...], preferred_element_type=jnp.float32)
    h = h + b1_ref[...]
    h_ref[...] = h
    s_ref[0] = jnp.sum(h, axis=0, keepdims=True)
    ss_ref[0] = jnp.sum(h * h, axis=0, keepdims=True)


def _mlp2_body(n_rows, h_ref, s_ref, ss_ref, g_ref, be_ref, w2_ref, b2_ref,
               o_ref):
    mean = jnp.sum(s_ref[:, 0, :], axis=0, keepdims=True) / n_rows
    meansq = jnp.sum(ss_ref[:, 0, :], axis=0, keepdims=True) / n_rows
    var = meansq - mean * mean
    h = (h_ref[...] - mean) / jnp.sqrt(var + 1e-5) * g_ref[...] + be_ref[...]
    h = jnp.maximum(h, 0.0)
    o = jnp.dot(h, w2_ref[...], preferred_element_type=jnp.float32)
    o_ref[...] = o + b2_ref[...]


def _mlp(x2d, W1, b1, gamma, beta, W2, b2):
    n, cin = x2d.shape
    dout = W2.shape[1]
    bm = 1024
    g = n // bm
    h, s, ss = pl.pallas_call(
        _mlp1_body,
        grid=(g,),
        in_specs=[
            pl.BlockSpec((bm, cin), lambda i: (i, 0)),
            pl.BlockSpec((cin, dout), lambda i: (0, 0)),
            pl.BlockSpec((1, dout), lambda i: (0, 0)),
        ],
        out_specs=[
            pl.BlockSpec((bm, dout), lambda i: (i, 0)),
            pl.BlockSpec((1, 1, dout), lambda i: (i, 0, 0)),
            pl.BlockSpec((1, 1, dout), lambda i: (i, 0, 0)),
        ],
        out_shape=[
            jax.ShapeDtypeStruct((n, dout), jnp.float32),
            jax.ShapeDtypeStruct((g, 1, dout), jnp.float32),
            jax.ShapeDtypeStruct((g, 1, dout), jnp.float32),
        ],
    )(x2d, W1, b1.reshape(1, -1))
    return pl.pallas_call(
        functools.partial(_mlp2_body, float(n)),
        grid=(g,),
        in_specs=[
            pl.BlockSpec((bm, dout), lambda i: (i, 0)),
            pl.BlockSpec((g, 1, dout), lambda i: (0, 0, 0)),
            pl.BlockSpec((g, 1, dout), lambda i: (0, 0, 0)),
            pl.BlockSpec((1, dout), lambda i: (0, 0)),
            pl.BlockSpec((1, dout), lambda i: (0, 0)),
            pl.BlockSpec((dout, dout), lambda i: (0, 0)),
            pl.BlockSpec((1, dout), lambda i: (0, 0)),
        ],
        out_specs=pl.BlockSpec((bm, dout), lambda i: (i, 0)),
        out_shape=jax.ShapeDtypeStruct((n, dout), jnp.float32),
    )(h, s, ss, gamma.reshape(1, -1), beta.reshape(1, -1), W2,
      b2.reshape(1, -1))


_FMAX = 3.4028235e38


def _knn_body(ns_total, pd_ref, ps_ref, idx_ref, w_ref):
    b = pl.program_id(0)
    q = pd_ref[0]
    r = ps_ref[0]
    ns = r.shape[0]
    cross = lax.dot_general(q, r, (((1,), (1,)), ((), ())),
                            preferred_element_type=jnp.float32)
    q2 = jnp.sum(q * q, axis=1, keepdims=True)
    r2 = jnp.sum(r * r, axis=1, keepdims=True).T
    d2 = jnp.maximum(q2 + r2 - 2.0 * cross, 0.0)
    iota = lax.broadcasted_iota(jnp.int32, d2.shape, 1)
    idxs = []
    dists = []
    for _ in range(3):
        m = jnp.min(d2, axis=1, keepdims=True)
        cand = jnp.where(d2 == m, iota, jnp.int32(ns))
        ij = jnp.min(cand, axis=1, keepdims=True)
        d2 = jnp.where(iota == ij, _FMAX, d2)
        idxs.append(ij)
        dists.append(m)
    recips = [1.0 / (d + 1e-8) for d in dists]
    norm = recips[0] + recips[1] + recips[2]
    base = b * ns_total
    for j in range(3):
        idx_ref[0, j, :] = (idxs[j] + base)[:, 0]
        w_ref[0, j, :] = (recips[j] / norm)[:, 0]


def _knn(pd8, ps8):
    B, ND, _ = pd8.shape
    NS = ps8.shape[1]
    BQ = 1024
    grid = (B, ND // BQ)
    return pl.pallas_call(
        functools.partial(_knn_body, NS),
        grid=grid,
        in_specs=[
            pl.BlockSpec((1, BQ, 8), lambda b, q: (b, q, 0)),
            pl.BlockSpec((1, NS, 8), lambda b, q: (b, 0, 0)),
        ],
        out_specs=[
            pl.BlockSpec((1, 3, BQ), lambda b, q: (b, 0, q)),
            pl.BlockSpec((1, 3, BQ), lambda b, q: (b, 0, q)),
        ],
        out_shape=[
            jax.ShapeDtypeStruct((B, 3, ND), jnp.int32),
            jax.ShapeDtypeStruct((B, 3, ND), jnp.float32),
        ],
    )(pd8, ps8)


_T = 16
_D = 512
_CCH = _D // 16


def _interp_sc(table, gidx_i, wexp_i, total_q):
    info = plsc.get_sparse_core_info()
    nw = info.num_cores * info.num_subcores
    q_per_w = total_q // nw
    n_chunks = q_per_w // _T
    mesh = plsc.VectorSubcoreMesh(core_axis_name="c", subcore_axis_name="s")

    @functools.partial(
        pl.kernel,
        mesh=mesh,
        out_type=jax.ShapeDtypeStruct((total_q, _D), jnp.float32),
        scratch_types=[
            pltpu.VMEM((2, 3 * _T), jnp.int32),
            pltpu.VMEM((2, 3 * _T * 16), jnp.float32),
            pltpu.VMEM((2, 3 * _T, _D), jnp.float32),
            pltpu.VMEM((2, _T, _D), jnp.float32),
            pltpu.SemaphoreType.DMA((2,)),
            pltpu.SemaphoreType.DMA((2,)),
            pltpu.SemaphoreType.DMA((2,)),
            pltpu.SemaphoreType.DMA((2,)),
        ],
    )
    def body(tab_hbm, idx_hbm, w_hbm, out_hbm,
             idx_v, w_v, rows_v, acc_v, sem_i, sem_w, sem_g, sem_o):
        wid = lax.axis_index("s") * info.num_cores + lax.axis_index("c")
        qbase = wid * q_per_w

        def issue_iw(t, p):
            r0 = pl.multiple_of((qbase + t * _T) * 3, 8)
            w0 = pl.multiple_of((qbase + t * _T) * 48, 8)
            ci = pltpu.make_async_copy(idx_hbm.at[pl.ds(r0, 3 * _T)],
                                       idx_v.at[p], sem_i.at[p])
            cw = pltpu.make_async_copy(w_hbm.at[pl.ds(w0, 48 * _T)],
                                       w_v.at[p], sem_w.at[p])
            return ci, cw

        class _GatherSet:
            def __init__(self, p):
                self.cps = [
                    pltpu.make_async_copy(
                        tab_hbm.at[idx_v.at[p, pl.ds(16 * k, 16)]],
                        rows_v.at[p, pl.ds(16 * k, 16)], sem_g.at[p])
                    for k in range(3)]
            def start(self):
                for cp in self.cps:
                    cp.start()
            def wait(self):
                for cp in self.cps:
                    cp.wait()

        def issue_gather(p):
            return _GatherSet(p)

        def issue_out(t, p):
            g0 = pl.multiple_of(qbase + t * _T, 8)
            return pltpu.make_async_copy(acc_v.at[p],
                                         out_hbm.at[pl.ds(g0, _T)],
                                         sem_o.at[p])

        ci, cw = issue_iw(0, 0)
        ci.start(); cw.start(); ci.wait(); cw.wait()
        issue_gather(0).start()
        ci, cw = issue_iw(1, 1)
        ci.start(); cw.start()

        def step(t, p):
            pn = 1 - p
            @pl.when(t + 1 < n_chunks)
            def _():
                ci, cw = issue_iw(t + 1, pn)
                ci.wait(); cw.wait()
                issue_gather(pn).start()
            issue_gather(p).wait()
            @pl.when(t >= 2)
            def _():
                issue_out(t - 2, p).wait()
            for q in range(_T):
                ws = [w_v[p, pl.ds((q * 3 + j) * 16, 16)] for j in range(3)]
                for c in range(_CCH):
                    cs = pl.ds(c * 16, 16)
                    acc_v[p, q, cs] = (rows_v[p, q * 3 + 0, cs] * ws[0]
                                       + rows_v[p, q * 3 + 1, cs] * ws[1]
                                       + rows_v[p, q * 3 + 2, cs] * ws[2])
            @pl.when(t + 2 < n_chunks)
            def _():
                ci, cw = issue_iw(t + 2, p)
                ci.start(); cw.start()
            issue_out(t, p).start()

        def pair(i, carry):
            step(2 * i, 0)
            step(2 * i + 1, 1)
            return carry

        lax.fori_loop(0, n_chunks // 2, pair, 0)
        issue_out(n_chunks - 2, 0).wait()
        issue_out(n_chunks - 1, 1).wait()

    return body(table, gidx_i, wexp_i)


def kernel(x, p_sparse, p_dense, W1, b1, gamma, beta, W2, b2):
    B, NS, C = x.shape
    ND = p_dense.shape[1]
    dout = W2.shape[1]
    x2 = _mlp(x.reshape(B * NS, C), W1, b1, gamma, beta, W2, b2)
    pad = lambda p: jnp.concatenate(
        [p, jnp.zeros(p.shape[:-1] + (5,), p.dtype)], axis=-1)
    pd8, ps8 = pad(p_dense), pad(p_sparse)
    outs = []
    for b in range(B):
        gidx, wts = _knn(pd8[b:b + 1], ps8[b:b + 1])
        gidx_i = (jnp.transpose(gidx, (0, 2, 1)) + b * NS).reshape(-1)
        wexp_i = jnp.broadcast_to(
            jnp.transpose(wts, (0, 2, 1))[..., None],
            (1, ND, 3, 16)).reshape(-1)
        outs.append(_interp_sc(x2, gidx_i, wexp_i, ND))
    out = jnp.concatenate(outs, axis=0)
    return out.reshape(B, ND, dout), p_dense


# speedup vs baseline: 1.0110x; 1.0110x over previous
import functools

import jax
import jax.numpy as jnp
from jax import lax
from jax.experimental import pallas as pl
from jax.experimental.pallas import tpu as pltpu
from jax.experimental.pallas import tpu_sc as plsc


def _mlp_body(x_ref, w1_ref, b1_ref, g_ref, be_ref, w2_ref, b2_ref, o_ref):
    h = jnp.dot(x_ref[...], w1_ref[...], preferred_element_type=jnp.float32)
    h = h + b1_ref[...]
    mean = jnp.mean(h, axis=0, keepdims=True)
    var = jnp.mean((h - mean) ** 2, axis=0, keepdims=True)
    h = (h - mean) / jnp.sqrt(var + 1e-5) * g_ref[...] + be_ref[...]
    h = jnp.maximum(h, 0.0)
    o = jnp.dot(h, w2_ref[...], preferred_element_type=jnp.float32)
    o_ref[...] = o + b2_ref[...]


def _mlp(x2d, W1, b1, gamma, beta, W2, b2):
    n, _ = x2d.shape
    dout = W2.shape[1]
    return pl.pallas_call(
        _mlp_body,
        out_shape=jax.ShapeDtypeStruct((n, dout), jnp.float32),
    )(x2d, W1, b1.reshape(1, -1), gamma.reshape(1, -1), beta.reshape(1, -1),
      W2, b2.reshape(1, -1))


_FMAX = 3.4028235e38


def _knn_body(ns_total, pd_ref, ps_ref, idx_ref, w_ref):
    b = pl.program_id(0)
    q = pd_ref[0]
    r = ps_ref[0]
    ns = r.shape[0]
    cross = lax.dot_general(q, r, (((1,), (1,)), ((), ())),
                            preferred_element_type=jnp.float32)
    q2 = jnp.sum(q * q, axis=1, keepdims=True)
    r2 = jnp.sum(r * r, axis=1, keepdims=True).T
    d2 = jnp.maximum(q2 + r2 - 2.0 * cross, 0.0)
    iota = lax.broadcasted_iota(jnp.int32, d2.shape, 1)
    idxs = []
    dists = []
    for _ in range(3):
        m = jnp.min(d2, axis=1, keepdims=True)
        cand = jnp.where(d2 == m, iota, jnp.int32(ns))
        ij = jnp.min(cand, axis=1, keepdims=True)
        d2 = jnp.where(iota == ij, _FMAX, d2)
        idxs.append(ij)
        dists.append(m)
    recips = [1.0 / (d + 1e-8) for d in dists]
    norm = recips[0] + recips[1] + recips[2]
    base = b * ns_total
    for j in range(3):
        idx_ref[0, j, :] = (idxs[j] + base)[:, 0]
        w_ref[0, j, :] = (recips[j] / norm)[:, 0]


def _knn(pd8, ps8):
    B, ND, _ = pd8.shape
    NS = ps8.shape[1]
    BQ = 1024
    grid = (B, ND // BQ)
    return pl.pallas_call(
        functools.partial(_knn_body, NS),
        grid=grid,
        in_specs=[
            pl.BlockSpec((1, BQ, 8), lambda b, q: (b, q, 0)),
            pl.BlockSpec((1, NS, 8), lambda b, q: (b, 0, 0)),
        ],
        out_specs=[
            pl.BlockSpec((1, 3, BQ), lambda b, q: (b, 0, q)),
            pl.BlockSpec((1, 3, BQ), lambda b, q: (b, 0, q)),
        ],
        out_shape=[
            jax.ShapeDtypeStruct((B, 3, ND), jnp.int32),
            jax.ShapeDtypeStruct((B, 3, ND), jnp.float32),
        ],
    )(pd8, ps8)


_T = 16
_D = 512
_CCH = _D // 16


def _interp_sc(table, gidx_i, wexp_i, total_q):
    info = plsc.get_sparse_core_info()
    nw = info.num_cores * info.num_subcores
    q_per_w = total_q // nw
    n_chunks = q_per_w // _T
    mesh = plsc.VectorSubcoreMesh(core_axis_name="c", subcore_axis_name="s")

    @functools.partial(
        pl.kernel,
        mesh=mesh,
        out_type=jax.ShapeDtypeStruct((total_q, _D), jnp.float32),
        scratch_types=[
            pltpu.VMEM((2, 3 * _T), jnp.int32),
            pltpu.VMEM((2, 3 * _T * 16), jnp.float32),
            pltpu.VMEM((2, 3 * _T, _D), jnp.float32),
            pltpu.VMEM((2, _T, _D), jnp.float32),
            pltpu.SemaphoreType.DMA((2,)),
            pltpu.SemaphoreType.DMA((2,)),
            pltpu.SemaphoreType.DMA((2,)),
            pltpu.SemaphoreType.DMA((2,)),
        ],
    )
    def body(tab_hbm, idx_hbm, w_hbm, out_hbm,
             idx_v, w_v, rows_v, acc_v, sem_i, sem_w, sem_g, sem_o):
        wid = lax.axis_index("s") * info.num_cores + lax.axis_index("c")
        qbase = wid * q_per_w

        def issue_iw(t, p):
            r0 = pl.multiple_of((qbase + t * _T) * 3, 8)
            w0 = pl.multiple_of((qbase + t * _T) * 48, 8)
            ci = pltpu.make_async_copy(idx_hbm.at[pl.ds(r0, 3 * _T)],
                                       idx_v.at[p], sem_i.at[p])
            cw = pltpu.make_async_copy(w_hbm.at[pl.ds(w0, 48 * _T)],
                                       w_v.at[p], sem_w.at[p])
            return ci, cw

        class _GatherSet:
            def __init__(self, p):
                self.cps = [
                    pltpu.make_async_copy(
                        tab_hbm.at[idx_v.at[p, pl.ds(16 * k, 16)]],
                        rows_v.at[p, pl.ds(16 * k, 16)], sem_g.at[p])
                    for k in range(3)]
            def start(self):
                for cp in self.cps:
                    cp.start()
            def wait(self):
                for cp in self.cps:
                    cp.wait()

        def issue_gather(p):
            return _GatherSet(p)

        def issue_out(t, p):
            g0 = pl.multiple_of(qbase + t * _T, 8)
            return pltpu.make_async_copy(acc_v.at[p],
                                         out_hbm.at[pl.ds(g0, _T)],
                                         sem_o.at[p])

        ci, cw = issue_iw(0, 0)
        ci.start(); cw.start(); ci.wait(); cw.wait()
        issue_gather(0).start()
        ci, cw = issue_iw(1, 1)
        ci.start(); cw.start()

        def step(t, p):
            pn = 1 - p
            @pl.when(t + 1 < n_chunks)
            def _():
                ci, cw = issue_iw(t + 1, pn)
                ci.wait(); cw.wait()
                issue_gather(pn).start()
            issue_gather(p).wait()
            @pl.when(t >= 2)
            def _():
                issue_out(t - 2, p).wait()
            for q in range(_T):
                ws = [w_v[p, pl.ds((q * 3 + j) * 16, 16)] for j in range(3)]
                for c in range(_CCH):
                    cs = pl.ds(c * 16, 16)
                    acc_v[p, q, cs] = (rows_v[p, q * 3 + 0, cs] * ws[0]
                                       + rows_v[p, q * 3 + 1, cs] * ws[1]
                                       + rows_v[p, q * 3 + 2, cs] * ws[2])
            @pl.when(t + 2 < n_chunks)
            def _():
                ci, cw = issue_iw(t + 2, p)
                ci.start(); cw.start()
            issue_out(t, p).start()

        def pair(i, carry):
            step(2 * i, 0)
            step(2 * i + 1, 1)
            return carry

        lax.fori_loop(0, n_chunks // 2, pair, 0)
        issue_out(n_chunks - 2, 0).wait()
        issue_out(n_chunks - 1, 1).wait()

    return body(table, gidx_i, wexp_i)


def kernel(x, p_sparse, p_dense, W1, b1, gamma, beta, W2, b2):
    B, NS, C = x.shape
    ND = p_dense.shape[1]
    dout = W2.shape[1]
    x2 = _mlp(x.reshape(B * NS, C), W1, b1, gamma, beta, W2, b2)
    pad = lambda p: jnp.concatenate(
        [p, jnp.zeros(p.shape[:-1] + (5,), p.dtype)], axis=-1)
    pd8, ps8 = pad(p_dense), pad(p_sparse)
    outs = []
    for b in range(B):
        gidx, wts = _knn(pd8[b:b + 1], ps8[b:b + 1])
        gidx_i = (jnp.transpose(gidx, (0, 2, 1)) + b * NS).reshape(-1)
        wexp_i = jnp.broadcast_to(
            jnp.transpose(wts, (0, 2, 1))[..., None],
            (1, ND, 3, 16)).reshape(-1)
        outs.append(_interp_sc(x2, gidx_i, wexp_i, ND))
    out = jnp.concatenate(outs, axis=0)
    return out.reshape(B, ND, dout), p_dense
